# baseline (speedup 1.0000x reference)
import functools

import jax
import jax.numpy as jnp
from jax import lax
from jax.experimental import pallas as pl
from jax.experimental.pallas import tpu as pltpu
from jax.experimental.pallas import tpu_sc as plsc

N = 10000
NP = 10240
E = 320000
H = 128

NC = 2
NS = 16
CHUNK = 112
NCH0 = 144
NCH1 = 36
EP = NS * CHUNK * (NCH0 + NCH1)
ROWS_PER_TILE = NP // NS
ZR = 16
NCOPY = ROWS_PER_TILE // ZR

_mesh = plsc.VectorSubcoreMesh(core_axis_name="c", subcore_axis_name="s")


def _fill_rows(buf, nrows, val):
    v = jnp.full((16,), val, jnp.float32)

    def row(r, carry):
        for j in range(H // 16):
            buf[r, pl.ds(16 * j, 16)] = v
        return carry

    lax.fori_loop(0, nrows, row, 0)


def _fill_vec(buf, n, val):
    v = jnp.full((16,), val, jnp.float32)

    def step(i, carry):
        buf[pl.ds(i * 16, 16)] = v
        return carry

    lax.fori_loop(0, n // 16, step, 0)


def _make_seg(with_degt):
    out_type = [jax.ShapeDtypeStruct((NC * NP, H), jnp.float32)]
    scratch = [
        pltpu.VMEM((CHUNK,), jnp.int32),
        pltpu.VMEM((CHUNK,), jnp.int32),
        pltpu.VMEM((CHUNK,), jnp.int32),
        pltpu.VMEM((CHUNK,), jnp.int32),
        pltpu.VMEM((CHUNK, H), jnp.float32),
        pltpu.VMEM((CHUNK, H), jnp.float32),
        pltpu.VMEM((ZR, H), jnp.float32),
        pltpu.VMEM_SHARED((NP, H), jnp.float32),
        pltpu.SemaphoreType.DMA,
        pltpu.SemaphoreType.DMA,
        pltpu.SemaphoreType.DMA,
        pltpu.SemaphoreType.DMA,
    ]
    if with_degt:
        out_type += [
            jax.ShapeDtypeStruct((NC * NP,), jnp.float32),
            jax.ShapeDtypeStruct((NC * NP,), jnp.float32),
        ]
        scratch += [
            pltpu.VMEM((CHUNK,), jnp.float32),
            pltpu.VMEM((CHUNK,), jnp.float32),
            pltpu.VMEM((CHUNK,), jnp.float32),
            pltpu.VMEM((ROWS_PER_TILE,), jnp.float32),
            pltpu.VMEM_SHARED((NP,), jnp.float32),
            pltpu.VMEM_SHARED((NP,), jnp.float32),
        ]

    def body(*refs):
        if with_degt:
            (src_hbm, dst_hbm, h_hbm, ea_hbm,
             s_out, deg_out, t_out,
             src0, src1, dst0, dst1, rows0, rows1, zbuf, acc_sh,
             lsem0, lsem1, gsem0, gsem1,
             ea0, ea1, ones_v, vbuf, dacc_sh, tacc_sh) = refs
        else:
            (src_hbm, dst_hbm, h_hbm,
             s_out,
             src0, src1, dst0, dst1, rows0, rows1, zbuf, acc_sh,
             lsem0, lsem1, gsem0, gsem1) = refs
            ea_hbm = ea0 = ea1 = ones_v = vbuf = dacc_sh = tacc_sh = None

        srcb = (src0, src1)
        dstb = (dst0, dst1)
        rowsb = (rows0, rows1)
        lsem = (lsem0, lsem1)
        gsem = (gsem0, gsem1)
        eab = (ea0, ea1)

        c = lax.axis_index("c")
        s = lax.axis_index("s")
        r0 = s * ROWS_PER_TILE
        nch = jnp.where(c == 0, NCH0, NCH1)
        base = jnp.where(c == 0, s * (NCH0 * CHUNK),
                         NS * NCH0 * CHUNK + s * (NCH1 * CHUNK))

        def chunk_off(ci):
            ci = jnp.where(ci >= nch, ci - nch, ci)
            return base + ci * CHUNK

        def issue_loads(ci, b):
            off = chunk_off(ci)
            h1 = pltpu.async_copy(src_hbm.at[pl.ds(off, CHUNK)], srcb[b], lsem[b])
            h2 = pltpu.async_copy(dst_hbm.at[pl.ds(off, CHUNK)], dstb[b], lsem[b])
            hs = [h1, h2]
            if with_degt:
                hs.append(pltpu.async_copy(ea_hbm.at[pl.ds(off, CHUNK)], eab[b], lsem[b]))
            return hs

        def wait_loads(ci, b):
            off = chunk_off(ci)
            pltpu.make_async_copy(src_hbm.at[pl.ds(off, CHUNK)], srcb[b], lsem[b]).wait()
            pltpu.make_async_copy(dst_hbm.at[pl.ds(off, CHUNK)], dstb[b], lsem[b]).wait()
            if with_degt:
                pltpu.make_async_copy(ea_hbm.at[pl.ds(off, CHUNK)], eab[b], lsem[b]).wait()

        def issue_gather(b):
            pltpu.async_copy(h_hbm.at[srcb[b]], rowsb[b], gsem[b])

        def wait_gather(b):
            pltpu.make_async_copy(h_hbm.at[srcb[b]], rowsb[b], gsem[b]).wait()

        def scatter(b):
            pltpu.sync_copy(rowsb[b], acc_sh.at[dstb[b]], add=True)
            if with_degt:
                pltpu.sync_copy(ones_v, dacc_sh.at[dstb[b]], add=True)
                pltpu.sync_copy(eab[b], tacc_sh.at[dstb[b]], add=True)

        _fill_rows(zbuf, ZR, 0.0)
        for k in range(NCOPY):
            pltpu.sync_copy(zbuf, acc_sh.at[pl.ds(r0 + k * ZR, ZR)])
        if with_degt:
            _fill_vec(vbuf, ROWS_PER_TILE, 0.0)
            _fill_vec(ones_v, CHUNK, 1.0)
            pltpu.sync_copy(vbuf, dacc_sh.at[pl.ds(r0, ROWS_PER_TILE)])
            pltpu.sync_copy(vbuf, tacc_sh.at[pl.ds(r0, ROWS_PER_TILE)])
        plsc.subcore_barrier()

        issue_loads(0, 0)
        wait_loads(0, 0)
        issue_gather(0)
        issue_loads(1, 1)

        def step(i, carry):
            for b in (0, 1):
                ci = 2 * i + b
                wait_gather(b)
                scatter(b)
                issue_loads(ci + 2, b)
                wait_loads(ci + 1, 1 - b)
                issue_gather(1 - b)
            return carry

        lax.fori_loop(0, nch // 2, step, 0)
        wait_gather(0)
        wait_loads(nch + 1, 1)

        plsc.subcore_barrier()
        o0 = c * NP + r0
        for k in range(NCOPY):
            pltpu.sync_copy(acc_sh.at[pl.ds(r0 + k * ZR, ZR)], zbuf)
            pltpu.sync_copy(zbuf, s_out.at[pl.ds(o0 + k * ZR, ZR)])
        if with_degt:
            pltpu.sync_copy(dacc_sh.at[pl.ds(r0, ROWS_PER_TILE)], vbuf)
            pltpu.sync_copy(vbuf, deg_out.at[pl.ds(o0, ROWS_PER_TILE)])
            pltpu.sync_copy(tacc_sh.at[pl.ds(r0, ROWS_PER_TILE)], vbuf)
            pltpu.sync_copy(vbuf, t_out.at[pl.ds(o0, ROWS_PER_TILE)])

    if not with_degt:
        out_type = out_type[0]
    return pl.kernel(body, mesh=_mesh, out_type=out_type, scratch_types=scratch)


_seg_round0 = _make_seg(with_degt=True)
_seg_round1 = _make_seg(with_degt=False)


BN = 1000


def _tc_body(h_ref, sa_ref, sb_ref, dt_ref,
             At_ref, Bt_ref, bm_ref, we_ref,
             Wiht_ref, Whht_ref, bih_ref, bhh_ref, out_ref):
    h = h_ref[...]
    S = sa_ref[...] + sb_ref[...]
    deg = dt_ref[:, 0:1]
    t = dt_ref[:, 1:2]
    a = (deg * (jnp.dot(h, At_ref[...], preferred_element_type=jnp.float32)
                + bm_ref[...])
         + jnp.dot(S, Bt_ref[...], preferred_element_type=jnp.float32)
         + t * we_ref[...])
    gi = jnp.dot(a, Wiht_ref[...],
                 preferred_element_type=jnp.float32) + bih_ref[...]
    gh = jnp.dot(h, Whht_ref[...],
                 preferred_element_type=jnp.float32) + bhh_ref[...]
    r = jax.nn.sigmoid(gi[:, :H] + gh[:, :H])
    z = jax.nn.sigmoid(gi[:, H:2 * H] + gh[:, H:2 * H])
    n = jnp.tanh(gi[:, 2 * H:] + r * gh[:, 2 * H:])
    hn = (1.0 - z) * n + z * h
    out_ref[...] = jnp.where(deg > 0, hn, h)


def _tc_update(h, sa, sb, dt, Wm, bm, Wih, Whh, bih, bhh):
    At = jnp.transpose(Wm[:, :H])
    Bt = jnp.transpose(Wm[:, H:2 * H])
    we = Wm[:, 2 * H][None, :]
    Wiht = jnp.transpose(Wih)
    Whht = jnp.transpose(Whh)
    row = lambda i: (i, 0)
    fixed = lambda i: (0, 0)
    return pl.pallas_call(
        _tc_body,
        grid=(N // BN,),
        in_specs=[
            pl.BlockSpec((BN, H), row),
            pl.BlockSpec((BN, H), row),
            pl.BlockSpec((BN, H), row),
            pl.BlockSpec((BN, 128), row),
            pl.BlockSpec((H, 2 * H), fixed),
            pl.BlockSpec((H, 2 * H), fixed),
            pl.BlockSpec((1, 2 * H), fixed),
            pl.BlockSpec((1, 2 * H), fixed),
            pl.BlockSpec((2 * H, 3 * H), fixed),
            pl.BlockSpec((H, 3 * H), fixed),
            pl.BlockSpec((1, 3 * H), fixed),
            pl.BlockSpec((1, 3 * H), fixed),
        ],
        out_specs=pl.BlockSpec((BN, H), row),
        out_shape=jax.ShapeDtypeStruct((N, H), jnp.float32),
    )(h, sa, sb, dt, At, Bt, bm[None, :], we, Wiht, Whht,
      bih[None, :], bhh[None, :])


def kernel(x, edge_index, edge_attr,
           W_msg0, b_msg0, W_ih0, W_hh0, b_ih0, b_hh0,
           W_msg1, b_msg1, W_ih1, W_hh1, b_ih1, b_hh1):
    src = edge_index[0].astype(jnp.int32)
    dst = edge_index[1].astype(jnp.int32)
    pad = EP - E
    srcp = jnp.concatenate([src, jnp.zeros((pad,), jnp.int32)])
    pad_dst = N + (jnp.arange(pad, dtype=jnp.int32) % (NP - N))
    dstp = jnp.concatenate([dst, pad_dst])
    eap = jnp.concatenate([edge_attr[:, 0], jnp.zeros((pad,), jnp.float32)])

    s_p, deg_p, t_p = _seg_round0(srcp, dstp, x, eap)
    deg = deg_p[:N] + deg_p[NP:NP + N]
    t = t_p[:N] + t_p[NP:NP + N]
    dt = jnp.zeros((N, 128), jnp.float32)
    dt = dt.at[:, 0].set(deg).at[:, 1].set(t)

    h = _tc_update(x, s_p[:N], s_p[NP:NP + N], dt,
                   W_msg0, b_msg0, W_ih0, W_hh0, b_ih0, b_hh0)
    s_p1 = _seg_round1(srcp, dstp, h)
    h = _tc_update(h, s_p1[:N], s_p1[NP:NP + N], dt,
                   W_msg1, b_msg1, W_ih1, W_hh1, b_ih1, b_hh1)
    return h


# speedup vs baseline: 1.0603x; 1.0603x over previous
import functools

import jax
import jax.numpy as jnp
from jax import lax
from jax.experimental import pallas as pl
from jax.experimental.pallas import tpu as pltpu
from jax.experimental.pallas import tpu_sc as plsc

N = 10000
NP = 10240
E = 320000
H = 128

NC = 2
NS = 16
CHUNK = 112
NCH0 = 132
NCH1 = 48
EP = NS * CHUNK * (NCH0 + NCH1)
ROWS_PER_TILE = NP // NS
ZR = 16
NCOPY = ROWS_PER_TILE // ZR

_mesh = plsc.VectorSubcoreMesh(core_axis_name="c", subcore_axis_name="s")


def _fill_rows(buf, nrows, val):
    v = jnp.full((16,), val, jnp.float32)

    def row(r, carry):
        for j in range(H // 16):
            buf[r, pl.ds(16 * j, 16)] = v
        return carry

    lax.fori_loop(0, nrows, row, 0)


def _fill_vec(buf, n, val):
    v = jnp.full((16,), val, jnp.float32)

    def step(i, carry):
        buf[pl.ds(i * 16, 16)] = v
        return carry

    lax.fori_loop(0, n // 16, step, 0)


def _make_seg(with_degt):
    out_type = [jax.ShapeDtypeStruct((NC * NP, H), jnp.float32)]
    scratch = [
        pltpu.VMEM((CHUNK,), jnp.int32),
        pltpu.VMEM((CHUNK,), jnp.int32),
        pltpu.VMEM((CHUNK,), jnp.int32),
        pltpu.VMEM((CHUNK,), jnp.int32),
        pltpu.VMEM((CHUNK, H), jnp.float32),
        pltpu.VMEM((CHUNK, H), jnp.float32),
        pltpu.VMEM((ZR, H), jnp.float32),
        pltpu.VMEM_SHARED((NP, H), jnp.float32),
        pltpu.SemaphoreType.DMA,
        pltpu.SemaphoreType.DMA,
        pltpu.SemaphoreType.DMA,
        pltpu.SemaphoreType.DMA,
    ]
    if with_degt:
        out_type += [
            jax.ShapeDtypeStruct((NC * NP,), jnp.float32),
            jax.ShapeDtypeStruct((NC * NP,), jnp.float32),
        ]
        scratch += [
            pltpu.VMEM((CHUNK,), jnp.float32),
            pltpu.VMEM((CHUNK,), jnp.float32),
            pltpu.VMEM((CHUNK,), jnp.float32),
            pltpu.VMEM((ROWS_PER_TILE,), jnp.float32),
            pltpu.VMEM_SHARED((NP,), jnp.float32),
            pltpu.VMEM_SHARED((NP,), jnp.float32),
        ]

    def body(*refs):
        if with_degt:
            (src_hbm, dst_hbm, h_hbm, ea_hbm,
             s_out, deg_out, t_out,
             src0, src1, dst0, dst1, rows0, rows1, zbuf, acc_sh,
             lsem0, lsem1, gsem0, gsem1,
             ea0, ea1, ones_v, vbuf, dacc_sh, tacc_sh) = refs
        else:
            (src_hbm, dst_hbm, h_hbm,
             s_out,
             src0, src1, dst0, dst1, rows0, rows1, zbuf, acc_sh,
             lsem0, lsem1, gsem0, gsem1) = refs
            ea_hbm = ea0 = ea1 = ones_v = vbuf = dacc_sh = tacc_sh = None

        srcb = (src0, src1)
        dstb = (dst0, dst1)
        rowsb = (rows0, rows1)
        lsem = (lsem0, lsem1)
        gsem = (gsem0, gsem1)
        eab = (ea0, ea1)

        c = lax.axis_index("c")
        s = lax.axis_index("s")
        r0 = s * ROWS_PER_TILE
        nch = jnp.where(c == 0, NCH0, NCH1)
        base = jnp.where(c == 0, s * (NCH0 * CHUNK),
                         NS * NCH0 * CHUNK + s * (NCH1 * CHUNK))

        def chunk_off(ci):
            ci = jnp.where(ci >= nch, ci - nch, ci)
            return base + ci * CHUNK

        def issue_loads(ci, b):
            off = chunk_off(ci)
            h1 = pltpu.async_copy(src_hbm.at[pl.ds(off, CHUNK)], srcb[b], lsem[b])
            h2 = pltpu.async_copy(dst_hbm.at[pl.ds(off, CHUNK)], dstb[b], lsem[b])
            hs = [h1, h2]
            if with_degt:
                hs.append(pltpu.async_copy(ea_hbm.at[pl.ds(off, CHUNK)], eab[b], lsem[b]))
            return hs

        def wait_loads(ci, b):
            off = chunk_off(ci)
            pltpu.make_async_copy(src_hbm.at[pl.ds(off, CHUNK)], srcb[b], lsem[b]).wait()
            pltpu.make_async_copy(dst_hbm.at[pl.ds(off, CHUNK)], dstb[b], lsem[b]).wait()
            if with_degt:
                pltpu.make_async_copy(ea_hbm.at[pl.ds(off, CHUNK)], eab[b], lsem[b]).wait()

        def issue_gather(b):
            pltpu.async_copy(h_hbm.at[srcb[b]], rowsb[b], gsem[b])

        def wait_gather(b):
            pltpu.make_async_copy(h_hbm.at[srcb[b]], rowsb[b], gsem[b]).wait()

        def scatter(b):
            pltpu.sync_copy(rowsb[b], acc_sh.at[dstb[b]], add=True)
            if with_degt:
                pltpu.sync_copy(ones_v, dacc_sh.at[dstb[b]], add=True)
                pltpu.sync_copy(eab[b], tacc_sh.at[dstb[b]], add=True)

        _fill_rows(zbuf, ZR, 0.0)
        for k in range(NCOPY):
            pltpu.sync_copy(zbuf, acc_sh.at[pl.ds(r0 + k * ZR, ZR)])
        if with_degt:
            _fill_vec(vbuf, ROWS_PER_TILE, 0.0)
            _fill_vec(ones_v, CHUNK, 1.0)
            pltpu.sync_copy(vbuf, dacc_sh.at[pl.ds(r0, ROWS_PER_TILE)])
            pltpu.sync_copy(vbuf, tacc_sh.at[pl.ds(r0, ROWS_PER_TILE)])
        plsc.subcore_barrier()

        issue_loads(0, 0)
        wait_loads(0, 0)
        issue_gather(0)
        issue_loads(1, 1)

        def step(i, carry):
            for b in (0, 1):
                ci = 2 * i + b
                wait_gather(b)
                scatter(b)
                issue_loads(ci + 2, b)
                wait_loads(ci + 1, 1 - b)
                issue_gather(1 - b)
            return carry

        lax.fori_loop(0, nch // 2, step, 0)
        wait_gather(0)
        wait_loads(nch + 1, 1)

        plsc.subcore_barrier()
        o0 = c * NP + r0
        for k in range(NCOPY):
            pltpu.sync_copy(acc_sh.at[pl.ds(r0 + k * ZR, ZR)], zbuf)
            pltpu.sync_copy(zbuf, s_out.at[pl.ds(o0 + k * ZR, ZR)])
        if with_degt:
            pltpu.sync_copy(dacc_sh.at[pl.ds(r0, ROWS_PER_TILE)], vbuf)
            pltpu.sync_copy(vbuf, deg_out.at[pl.ds(o0, ROWS_PER_TILE)])
            pltpu.sync_copy(tacc_sh.at[pl.ds(r0, ROWS_PER_TILE)], vbuf)
            pltpu.sync_copy(vbuf, t_out.at[pl.ds(o0, ROWS_PER_TILE)])

    if not with_degt:
        out_type = out_type[0]
    return pl.kernel(body, mesh=_mesh, out_type=out_type, scratch_types=scratch)


_seg_round0 = _make_seg(with_degt=True)
_seg_round1 = _make_seg(with_degt=False)


BN = 1000


def _tc_body(h_ref, sa_ref, sb_ref, dt_ref,
             At_ref, Bt_ref, bm_ref, we_ref,
             Wiht_ref, Whht_ref, bih_ref, bhh_ref, out_ref):
    h = h_ref[...]
    S = sa_ref[...] + sb_ref[...]
    deg = dt_ref[:, 0:1]
    t = dt_ref[:, 1:2]
    a = (deg * (jnp.dot(h, At_ref[...], preferred_element_type=jnp.float32)
                + bm_ref[...])
         + jnp.dot(S, Bt_ref[...], preferred_element_type=jnp.float32)
         + t * we_ref[...])
    gi = jnp.dot(a, Wiht_ref[...],
                 preferred_element_type=jnp.float32) + bih_ref[...]
    gh = jnp.dot(h, Whht_ref[...],
                 preferred_element_type=jnp.float32) + bhh_ref[...]
    r = jax.nn.sigmoid(gi[:, :H] + gh[:, :H])
    z = jax.nn.sigmoid(gi[:, H:2 * H] + gh[:, H:2 * H])
    n = jnp.tanh(gi[:, 2 * H:] + r * gh[:, 2 * H:])
    hn = (1.0 - z) * n + z * h
    out_ref[...] = jnp.where(deg > 0, hn, h)


def _tc_update(h, sa, sb, dt, Wm, bm, Wih, Whh, bih, bhh):
    At = jnp.transpose(Wm[:, :H])
    Bt = jnp.transpose(Wm[:, H:2 * H])
    we = Wm[:, 2 * H][None, :]
    Wiht = jnp.transpose(Wih)
    Whht = jnp.transpose(Whh)
    row = lambda i: (i, 0)
    fixed = lambda i: (0, 0)
    return pl.pallas_call(
        _tc_body,
        grid=(N // BN,),
        in_specs=[
            pl.BlockSpec((BN, H), row),
            pl.BlockSpec((BN, H), row),
            pl.BlockSpec((BN, H), row),
            pl.BlockSpec((BN, 128), row),
            pl.BlockSpec((H, 2 * H), fixed),
            pl.BlockSpec((H, 2 * H), fixed),
            pl.BlockSpec((1, 2 * H), fixed),
            pl.BlockSpec((1, 2 * H), fixed),
            pl.BlockSpec((2 * H, 3 * H), fixed),
            pl.BlockSpec((H, 3 * H), fixed),
            pl.BlockSpec((1, 3 * H), fixed),
            pl.BlockSpec((1, 3 * H), fixed),
        ],
        out_specs=pl.BlockSpec((BN, H), row),
        out_shape=jax.ShapeDtypeStruct((N, H), jnp.float32),
    )(h, sa, sb, dt, At, Bt, bm[None, :], we, Wiht, Whht,
      bih[None, :], bhh[None, :])


def kernel(x, edge_index, edge_attr,
           W_msg0, b_msg0, W_ih0, W_hh0, b_ih0, b_hh0,
           W_msg1, b_msg1, W_ih1, W_hh1, b_ih1, b_hh1):
    src = edge_index[0].astype(jnp.int32)
    dst = edge_index[1].astype(jnp.int32)
    pad = EP - E
    srcp = jnp.concatenate([src, jnp.zeros((pad,), jnp.int32)])
    pad_dst = N + (jnp.arange(pad, dtype=jnp.int32) % (NP - N))
    dstp = jnp.concatenate([dst, pad_dst])
    eap = jnp.concatenate([edge_attr[:, 0], jnp.zeros((pad,), jnp.float32)])

    s_p, deg_p, t_p = _seg_round0(srcp, dstp, x, eap)
    deg = deg_p[:N] + deg_p[NP:NP + N]
    t = t_p[:N] + t_p[NP:NP + N]
    dt = jnp.zeros((N, 128), jnp.float32)
    dt = dt.at[:, 0].set(deg).at[:, 1].set(t)

    h = _tc_update(x, s_p[:N], s_p[NP:NP + N], dt,
                   W_msg0, b_msg0, W_ih0, W_hh0, b_ih0, b_hh0)
    s_p1 = _seg_round1(srcp, dstp, h)
    h = _tc_update(h, s_p1[:N], s_p1[NP:NP + N], dt,
                   W_msg1, b_msg1, W_ih1, W_hh1, b_ih1, b_hh1)
    return h


# speedup vs baseline: 1.0881x; 1.0262x over previous
import functools

import jax
import jax.numpy as jnp
from jax import lax
from jax.experimental import pallas as pl
from jax.experimental.pallas import tpu as pltpu
from jax.experimental.pallas import tpu_sc as plsc

N = 10000
NP = 10240
E = 320000
H = 128

NC = 2
NS = 16
CHUNK = 112
NCH0 = 126
NCH1 = 54
EP = NS * CHUNK * (NCH0 + NCH1)
ROWS_PER_TILE = NP // NS
ZR = 16
NCOPY = ROWS_PER_TILE // ZR

_mesh = plsc.VectorSubcoreMesh(core_axis_name="c", subcore_axis_name="s")


def _fill_rows(buf, nrows, val):
    v = jnp.full((16,), val, jnp.float32)

    def row(r, carry):
        for j in range(H // 16):
            buf[r, pl.ds(16 * j, 16)] = v
        return carry

    lax.fori_loop(0, nrows, row, 0)


def _fill_vec(buf, n, val):
    v = jnp.full((16,), val, jnp.float32)

    def step(i, carry):
        buf[pl.ds(i * 16, 16)] = v
        return carry

    lax.fori_loop(0, n // 16, step, 0)


def _make_seg(with_degt):
    out_type = [jax.ShapeDtypeStruct((NC * NP, H), jnp.float32)]
    scratch = [
        pltpu.VMEM((CHUNK,), jnp.int32),
        pltpu.VMEM((CHUNK,), jnp.int32),
        pltpu.VMEM((CHUNK,), jnp.int32),
        pltpu.VMEM((CHUNK,), jnp.int32),
        pltpu.VMEM((CHUNK, H), jnp.float32),
        pltpu.VMEM((CHUNK, H), jnp.float32),
        pltpu.VMEM((ZR, H), jnp.float32),
        pltpu.VMEM_SHARED((NP, H), jnp.float32),
        pltpu.SemaphoreType.DMA,
        pltpu.SemaphoreType.DMA,
        pltpu.SemaphoreType.DMA,
        pltpu.SemaphoreType.DMA,
    ]
    if with_degt:
        out_type += [
            jax.ShapeDtypeStruct((NC * NP,), jnp.float32),
            jax.ShapeDtypeStruct((NC * NP,), jnp.float32),
        ]
        scratch += [
            pltpu.VMEM((CHUNK,), jnp.float32),
            pltpu.VMEM((CHUNK,), jnp.float32),
            pltpu.VMEM((CHUNK,), jnp.float32),
            pltpu.VMEM((ROWS_PER_TILE,), jnp.float32),
            pltpu.VMEM_SHARED((NP,), jnp.float32),
            pltpu.VMEM_SHARED((NP,), jnp.float32),
        ]

    def body(*refs):
        if with_degt:
            (src_hbm, dst_hbm, h_hbm, ea_hbm,
             s_out, deg_out, t_out,
             src0, src1, dst0, dst1, rows0, rows1, zbuf, acc_sh,
             lsem0, lsem1, gsem0, gsem1,
             ea0, ea1, ones_v, vbuf, dacc_sh, tacc_sh) = refs
        else:
            (src_hbm, dst_hbm, h_hbm,
             s_out,
             src0, src1, dst0, dst1, rows0, rows1, zbuf, acc_sh,
             lsem0, lsem1, gsem0, gsem1) = refs
            ea_hbm = ea0 = ea1 = ones_v = vbuf = dacc_sh = tacc_sh = None

        srcb = (src0, src1)
        dstb = (dst0, dst1)
        rowsb = (rows0, rows1)
        lsem = (lsem0, lsem1)
        gsem = (gsem0, gsem1)
        eab = (ea0, ea1)

        c = lax.axis_index("c")
        s = lax.axis_index("s")
        r0 = s * ROWS_PER_TILE
        nch = jnp.where(c == 0, NCH0, NCH1)
        base = jnp.where(c == 0, s * (NCH0 * CHUNK),
                         NS * NCH0 * CHUNK + s * (NCH1 * CHUNK))

        def chunk_off(ci):
            ci = jnp.where(ci >= nch, ci - nch, ci)
            return base + ci * CHUNK

        def issue_loads(ci, b):
            off = chunk_off(ci)
            h1 = pltpu.async_copy(src_hbm.at[pl.ds(off, CHUNK)], srcb[b], lsem[b])
            h2 = pltpu.async_copy(dst_hbm.at[pl.ds(off, CHUNK)], dstb[b], lsem[b])
            hs = [h1, h2]
            if with_degt:
                hs.append(pltpu.async_copy(ea_hbm.at[pl.ds(off, CHUNK)], eab[b], lsem[b]))
            return hs

        def wait_loads(ci, b):
            off = chunk_off(ci)
            pltpu.make_async_copy(src_hbm.at[pl.ds(off, CHUNK)], srcb[b], lsem[b]).wait()
            pltpu.make_async_copy(dst_hbm.at[pl.ds(off, CHUNK)], dstb[b], lsem[b]).wait()
            if with_degt:
                pltpu.make_async_copy(ea_hbm.at[pl.ds(off, CHUNK)], eab[b], lsem[b]).wait()

        def issue_gather(b):
            pltpu.async_copy(h_hbm.at[srcb[b]], rowsb[b], gsem[b])

        def wait_gather(b):
            pltpu.make_async_copy(h_hbm.at[srcb[b]], rowsb[b], gsem[b]).wait()

        def scatter(b):
            pltpu.sync_copy(rowsb[b], acc_sh.at[dstb[b]], add=True)
            if with_degt:
                pltpu.sync_copy(ones_v, dacc_sh.at[dstb[b]], add=True)
                pltpu.sync_copy(eab[b], tacc_sh.at[dstb[b]], add=True)

        _fill_rows(zbuf, ZR, 0.0)
        for k in range(NCOPY):
            pltpu.sync_copy(zbuf, acc_sh.at[pl.ds(r0 + k * ZR, ZR)])
        if with_degt:
            _fill_vec(vbuf, ROWS_PER_TILE, 0.0)
            _fill_vec(ones_v, CHUNK, 1.0)
            pltpu.sync_copy(vbuf, dacc_sh.at[pl.ds(r0, ROWS_PER_TILE)])
            pltpu.sync_copy(vbuf, tacc_sh.at[pl.ds(r0, ROWS_PER_TILE)])
        plsc.subcore_barrier()

        issue_loads(0, 0)
        wait_loads(0, 0)
        issue_gather(0)
        issue_loads(1, 1)

        def step(i, carry):
            for b in (0, 1):
                ci = 2 * i + b
                wait_gather(b)
                scatter(b)
                issue_loads(ci + 2, b)
                wait_loads(ci + 1, 1 - b)
                issue_gather(1 - b)
            return carry

        lax.fori_loop(0, nch // 2, step, 0)
        wait_gather(0)
        wait_loads(nch + 1, 1)

        plsc.subcore_barrier()
        o0 = c * NP + r0
        for k in range(NCOPY):
            pltpu.sync_copy(acc_sh.at[pl.ds(r0 + k * ZR, ZR)], zbuf)
            pltpu.sync_copy(zbuf, s_out.at[pl.ds(o0 + k * ZR, ZR)])
        if with_degt:
            pltpu.sync_copy(dacc_sh.at[pl.ds(r0, ROWS_PER_TILE)], vbuf)
            pltpu.sync_copy(vbuf, deg_out.at[pl.ds(o0, ROWS_PER_TILE)])
            pltpu.sync_copy(tacc_sh.at[pl.ds(r0, ROWS_PER_TILE)], vbuf)
            pltpu.sync_copy(vbuf, t_out.at[pl.ds(o0, ROWS_PER_TILE)])

    if not with_degt:
        out_type = out_type[0]
    return pl.kernel(body, mesh=_mesh, out_type=out_type, scratch_types=scratch)


_seg_round0 = _make_seg(with_degt=True)
_seg_round1 = _make_seg(with_degt=False)


BN = 1000


def _tc_body(h_ref, sa_ref, sb_ref, dt_ref,
             At_ref, Bt_ref, bm_ref, we_ref,
             Wiht_ref, Whht_ref, bih_ref, bhh_ref, out_ref):
    h = h_ref[...]
    S = sa_ref[...] + sb_ref[...]
    deg = dt_ref[:, 0:1]
    t = dt_ref[:, 1:2]
    a = (deg * (jnp.dot(h, At_ref[...], preferred_element_type=jnp.float32)
                + bm_ref[...])
         + jnp.dot(S, Bt_ref[...], preferred_element_type=jnp.float32)
         + t * we_ref[...])
    gi = jnp.dot(a, Wiht_ref[...],
                 preferred_element_type=jnp.float32) + bih_ref[...]
    gh = jnp.dot(h, Whht_ref[...],
                 preferred_element_type=jnp.float32) + bhh_ref[...]
    r = jax.nn.sigmoid(gi[:, :H] + gh[:, :H])
    z = jax.nn.sigmoid(gi[:, H:2 * H] + gh[:, H:2 * H])
    n = jnp.tanh(gi[:, 2 * H:] + r * gh[:, 2 * H:])
    hn = (1.0 - z) * n + z * h
    out_ref[...] = jnp.where(deg > 0, hn, h)


def _tc_update(h, sa, sb, dt, Wm, bm, Wih, Whh, bih, bhh):
    At = jnp.transpose(Wm[:, :H])
    Bt = jnp.transpose(Wm[:, H:2 * H])
    we = Wm[:, 2 * H][None, :]
    Wiht = jnp.transpose(Wih)
    Whht = jnp.transpose(Whh)
    row = lambda i: (i, 0)
    fixed = lambda i: (0, 0)
    return pl.pallas_call(
        _tc_body,
        grid=(N // BN,),
        in_specs=[
            pl.BlockSpec((BN, H), row),
            pl.BlockSpec((BN, H), row),
            pl.BlockSpec((BN, H), row),
            pl.BlockSpec((BN, 128), row),
            pl.BlockSpec((H, 2 * H), fixed),
            pl.BlockSpec((H, 2 * H), fixed),
            pl.BlockSpec((1, 2 * H), fixed),
            pl.BlockSpec((1, 2 * H), fixed),
            pl.BlockSpec((2 * H, 3 * H), fixed),
            pl.BlockSpec((H, 3 * H), fixed),
            pl.BlockSpec((1, 3 * H), fixed),
            pl.BlockSpec((1, 3 * H), fixed),
        ],
        out_specs=pl.BlockSpec((BN, H), row),
        out_shape=jax.ShapeDtypeStruct((N, H), jnp.float32),
    )(h, sa, sb, dt, At, Bt, bm[None, :], we, Wiht, Whht,
      bih[None, :], bhh[None, :])


def kernel(x, edge_index, edge_attr,
           W_msg0, b_msg0, W_ih0, W_hh0, b_ih0, b_hh0,
           W_msg1, b_msg1, W_ih1, W_hh1, b_ih1, b_hh1):
    src = edge_index[0].astype(jnp.int32)
    dst = edge_index[1].astype(jnp.int32)
    pad = EP - E
    srcp = jnp.concatenate([src, jnp.zeros((pad,), jnp.int32)])
    pad_dst = N + (jnp.arange(pad, dtype=jnp.int32) % (NP - N))
    dstp = jnp.concatenate([dst, pad_dst])
    eap = jnp.concatenate([edge_attr[:, 0], jnp.zeros((pad,), jnp.float32)])

    s_p, deg_p, t_p = _seg_round0(srcp, dstp, x, eap)
    deg = deg_p[:N] + deg_p[NP:NP + N]
    t = t_p[:N] + t_p[NP:NP + N]
    dt = jnp.zeros((N, 128), jnp.float32)
    dt = dt.at[:, 0].set(deg).at[:, 1].set(t)

    h = _tc_update(x, s_p[:N], s_p[NP:NP + N], dt,
                   W_msg0, b_msg0, W_ih0, W_hh0, b_ih0, b_hh0)
    s_p1 = _seg_round1(srcp, dstp, h)
    h = _tc_update(h, s_p1[:N], s_p1[NP:NP + N], dt,
                   W_msg1, b_msg1, W_ih1, W_hh1, b_ih1, b_hh1)
    return h
